# baseline (speedup 1.0000x reference)
import jax
import jax.numpy as jnp
from jax.experimental import pallas as pl

B, N, H = 16384, 1024, 256
TILE_B = 4096
LOG2E = 1.4426950408889634


def _retrieval_kernel(z_ref, mem_ref, out_ref):
    z = z_ref[...]
    mem = mem_ref[...]

    z_norm = z * jax.lax.rsqrt(jnp.maximum(jnp.sum(z * z, axis=1, keepdims=True), 1e-24))

    m_inv = jax.lax.rsqrt(jnp.maximum(jnp.sum(mem * mem, axis=1, keepdims=True), 1e-24))
    key = (mem * (m_inv * LOG2E)).astype(jnp.bfloat16)

    sim = jax.lax.dot_general(
        z_norm.astype(jnp.bfloat16), key,
        (((1,), (1,)), ((), ())),
        preferred_element_type=jnp.float32,
    )

    e = jnp.exp2(sim.astype(jnp.bfloat16))

    val_aug = jnp.concatenate(
        [mem.astype(jnp.bfloat16), jnp.ones((N, 128), dtype=jnp.bfloat16)], axis=1)
    acc = jnp.dot(e, val_aug, preferred_element_type=jnp.float32)
    out_ref[...] = acc[:, :H] * (1.0 / acc[:, H:H + 1])


def kernel(z, memory):
    return pl.pallas_call(
        _retrieval_kernel,
        grid=(B // TILE_B,),
        in_specs=[
            pl.BlockSpec((TILE_B, H), lambda i: (i, 0)),
            pl.BlockSpec((N, H), lambda i: (0, 0)),
        ],
        out_specs=pl.BlockSpec((TILE_B, H), lambda i: (i, 0)),
        out_shape=jax.ShapeDtypeStruct((B, H), jnp.float32),
    )(z, memory)


# speedup vs baseline: 1.3082x; 1.3082x over previous
import jax
import jax.numpy as jnp
from jax.experimental import pallas as pl

B, N, H = 16384, 1024, 256
TILE_B = 4096
LOG2E = 1.4426950408889634


def _retrieval_kernel(z_ref, mem_ref, out_ref):
    z = z_ref[...]
    mem = mem_ref[...]

    z_norm = z * jax.lax.rsqrt(jnp.maximum(jnp.sum(z * z, axis=1, keepdims=True), 1e-24))

    m_inv = jax.lax.rsqrt(jnp.maximum(jnp.sum(mem * mem, axis=1, keepdims=True), 1e-24))
    key = (mem * (m_inv * LOG2E)).astype(jnp.bfloat16)

    sim = jax.lax.dot_general(
        z_norm.astype(jnp.bfloat16), key,
        (((1,), (1,)), ((), ())),
        preferred_element_type=jnp.float32,
    )

    e = jnp.exp2(sim.astype(jnp.bfloat16))
    inv_sum = 1.0 / jnp.sum(e, axis=1, keepdims=True, dtype=jnp.float32)

    acc = jnp.dot(e, mem.astype(jnp.bfloat16), preferred_element_type=jnp.float32)
    out_ref[...] = acc * inv_sum


def kernel(z, memory):
    return pl.pallas_call(
        _retrieval_kernel,
        grid=(B // TILE_B,),
        in_specs=[
            pl.BlockSpec((TILE_B, H), lambda i: (i, 0)),
            pl.BlockSpec((N, H), lambda i: (0, 0)),
        ],
        out_specs=pl.BlockSpec((TILE_B, H), lambda i: (i, 0)),
        out_shape=jax.ShapeDtypeStruct((B, H), jnp.float32),
    )(z, memory)


# speedup vs baseline: 1.3271x; 1.0145x over previous
import jax
import jax.numpy as jnp
from jax.experimental import pallas as pl

B, N, H = 16384, 1024, 256
TILE_B = 4096
LOG2E = 1.4426950408889634


def _retrieval_kernel(z_ref, mem_ref, out_ref):
    z = z_ref[...]
    mem = mem_ref[...]

    z_norm = z * jax.lax.rsqrt(jnp.maximum(jnp.sum(z * z, axis=1, keepdims=True), 1e-24))

    m_inv = jax.lax.rsqrt(jnp.maximum(jnp.sum(mem * mem, axis=1, keepdims=True), 1e-24))
    key = (mem * (m_inv * LOG2E)).astype(jnp.bfloat16)

    sim = jax.lax.dot_general(
        z_norm.astype(jnp.bfloat16), key,
        (((1,), (1,)), ((), ())),
        preferred_element_type=jnp.float32,
    )

    e = jnp.exp2(sim.astype(jnp.bfloat16))

    p = e[:, :512] + e[:, 512:]
    p = p[:, :256] + p[:, 256:]
    p = p[:, :128] + p[:, 128:]
    inv_sum = 1.0 / jnp.sum(p, axis=1, keepdims=True, dtype=jnp.float32)

    acc = jnp.dot(e, mem.astype(jnp.bfloat16), preferred_element_type=jnp.float32)
    out_ref[...] = acc * inv_sum


def kernel(z, memory):
    return pl.pallas_call(
        _retrieval_kernel,
        grid=(B // TILE_B,),
        in_specs=[
            pl.BlockSpec((TILE_B, H), lambda i: (i, 0)),
            pl.BlockSpec((N, H), lambda i: (0, 0)),
        ],
        out_specs=pl.BlockSpec((TILE_B, H), lambda i: (i, 0)),
        out_shape=jax.ShapeDtypeStruct((B, H), jnp.float32),
    )(z, memory)
